# Initial kernel scaffold; baseline (speedup 1.0000x reference)
import jax
import jax.numpy as jnp
from jax.experimental import pallas as pl


def kernel(xh_atoms, xh_residues, t, mask_atoms, mask_residues, params):
    raise NotImplementedError("write your pallas kernel here")


# speedup vs baseline: 3.4781x; 3.4781x over previous
import functools

import jax
import jax.numpy as jnp
import numpy as np
from jax import lax
from jax.experimental import pallas as pl
from jax.experimental.pallas import tpu as pltpu
from jax.experimental.pallas import tpu_sc as plsc

N_ATOMS = 4000
N_RES = 2000
ATOM_NF = 16
RES_NF = 21
HIDDEN = 128
NLAYERS = 4
NHEADS = 8
NRBF = 32
KNN = 16
CUTOFF = 5.0
NBATCH = 32

N = N_ATOMS + N_RES
NP = 6144
E = NP * KNN
BLK = 256
GRID = NP // BLK

_START = float(np.exp(-CUTOFF))
_BETA = float(((2.0 / NRBF) * (1.0 - _START)) ** -2)
_MEANS = np.linspace(_START, 1.0, NRBF, dtype=np.float32)

F32 = jnp.float32


def _silu(x):
    return x * jax.nn.sigmoid(x)


def _enc_kernel(h_ref, w1_ref, b1_ref, w2_ref, b2_ref, we_ref, tv_ref,
                eb_ref, o_ref):
    h = h_ref[...]
    h1 = _silu(jnp.dot(h, w1_ref[...], preferred_element_type=F32) + b1_ref[...])
    h2 = jnp.dot(h1, w2_ref[...], preferred_element_type=F32) + b2_ref[...]
    h2t = jnp.concatenate([h2, jnp.full((h2.shape[0], 1), tv_ref[0, 0], F32)],
                          axis=1)
    o_ref[...] = jnp.dot(h2t, we_ref[...], preferred_element_type=F32) + eb_ref[...]


def _encode(h, w1, b1, w2, b2, we, tb, eb):
    rows, nf = h.shape
    grid = rows // 512
    full = lambda s: pl.BlockSpec(s, lambda i: (0, 0))
    return pl.pallas_call(
        _enc_kernel,
        grid=(grid,),
        in_specs=[
            pl.BlockSpec((512, nf), lambda i: (i, 0)),
            full((nf, HIDDEN)), full((1, HIDDEN)),
            full((HIDDEN, HIDDEN)), full((1, HIDDEN)),
            full((HIDDEN + 1, HIDDEN)),
            full((1, 1)), full((1, HIDDEN)),
        ],
        out_specs=pl.BlockSpec((512, HIDDEN), lambda i: (i, 0)),
        out_shape=jax.ShapeDtypeStruct((rows, HIDDEN), F32),
    )(h, w1, b1, w2, b2, we, tb, eb)


def _knn_kernel(pos_ref, posT_ref, sqr_ref, sqc_ref, br_ref, bc_ref, ew_ref,
                mean_ref, idx_ref, env_ref, u0_ref, u1_ref, u2_ref, ef_ref,
                d2_ref):
    pos_b = pos_ref[...]
    posT = posT_ref[...]
    sq_col = sqc_ref[...]
    sq_row = sqr_ref[...]
    d2 = sq_row + sq_col - 2.0 * jnp.dot(pos_b, posT,
                                         preferred_element_type=F32)
    d2 = jnp.maximum(d2, 0.0)
    same = br_ref[...] == bc_ref[...]
    d2_ref[...] = jnp.where(same, d2, 1e12)

    col = lax.broadcasted_iota(jnp.int32, (BLK, NP), 1)
    ew = ew_ref[...]
    means = mean_ref[...]
    for k in range(KNN):
        cur = d2_ref[...]
        m = jnp.min(cur, axis=1, keepdims=True)
        cand = jnp.where(cur == m, col, NP)
        j = jnp.min(cand, axis=1, keepdims=True)
        sel = col == j
        d2_ref[...] = jnp.where(sel, 1e30, cur)
        posj = [jnp.sum(jnp.where(sel, posT_ref[d:d + 1, :], 0.0),
                        axis=1, keepdims=True) for d in range(3)]
        dist = jnp.sqrt(jnp.maximum(m, 1e-12))
        env = jnp.where(dist < CUTOFF,
                        0.5 * (jnp.cos(jnp.pi * dist / CUTOFF) + 1.0), 0.0)
        idx_ref[:, k:k + 1] = j
        env_ref[:, k:k + 1] = env
        inv = 1.0 / (dist + 1e-8)
        u0_ref[:, k:k + 1] = (posj[0] - pos_b[:, 0:1]) * inv
        u1_ref[:, k:k + 1] = (posj[1] - pos_b[:, 1:2]) * inv
        u2_ref[:, k:k + 1] = (posj[2] - pos_b[:, 2:3]) * inv
        rbf = jnp.exp(-_BETA * (jnp.exp(-dist) - means) ** 2) * env
        ef_ref[k] = jnp.dot(rbf, ew, preferred_element_type=F32)


def _knn(pos_pad, posT, sq_pad, batch_col, edge_w):
    batch_row = batch_col.reshape(NP, 1)
    full = lambda s: pl.BlockSpec(s, lambda i: tuple(0 for _ in s))
    return pl.pallas_call(
        _knn_kernel,
        grid=(GRID,),
        in_specs=[
            pl.BlockSpec((BLK, 8), lambda i: (i, 0)),
            full((8, NP)),
            pl.BlockSpec((BLK, 1), lambda i: (i, 0)),
            full((1, NP)),
            pl.BlockSpec((BLK, 1), lambda i: (i, 0)),
            full((1, NP)),
            full((NRBF, HIDDEN)),
            full((1, NRBF)),
        ],
        out_specs=[
            pl.BlockSpec((BLK, KNN), lambda i: (i, 0)),
            pl.BlockSpec((BLK, KNN), lambda i: (i, 0)),
            pl.BlockSpec((BLK, KNN), lambda i: (i, 0)),
            pl.BlockSpec((BLK, KNN), lambda i: (i, 0)),
            pl.BlockSpec((BLK, KNN), lambda i: (i, 0)),
            pl.BlockSpec((KNN, BLK, HIDDEN), lambda i: (0, i, 0)),
        ],
        out_shape=[
            jax.ShapeDtypeStruct((NP, KNN), jnp.int32),
            jax.ShapeDtypeStruct((NP, KNN), F32),
            jax.ShapeDtypeStruct((NP, KNN), F32),
            jax.ShapeDtypeStruct((NP, KNN), F32),
            jax.ShapeDtypeStruct((NP, KNN), F32),
            jax.ShapeDtypeStruct((KNN, NP, HIDDEN), F32),
        ],
        scratch_shapes=[pltpu.VMEM((BLK, NP), F32)],
    )(pos_pad, posT, sq_pad.reshape(NP, 1), sq_pad.reshape(1, NP),
      batch_row, batch_col.reshape(1, NP), edge_w,
      jnp.linspace(_START, 1.0, NRBF).astype(F32).reshape(1, NRBF))


_SC_CHUNK = 128


def _sc_gather(idx_flat, tables):
    info = plsc.get_sparse_core_info()
    nw = info.num_cores * info.num_subcores
    per_w = E // nw
    nch = per_w // _SC_CHUNK
    mesh = plsc.VectorSubcoreMesh(core_axis_name="c", subcore_axis_name="s")
    dims = [t.shape[1] for t in tables]

    out_type = tuple(jax.ShapeDtypeStruct((E, d), F32) for d in dims)
    scratch = [pltpu.VMEM((_SC_CHUNK,), jnp.int32)]
    scratch += [pltpu.VMEM((_SC_CHUNK, d), F32) for d in dims]
    scratch += [pltpu.SemaphoreType.DMA for _ in dims]

    @functools.partial(pl.kernel, mesh=mesh, out_type=out_type,
                       scratch_types=scratch)
    def gat(*refs):
        nt = len(dims)
        idx_h = refs[0]
        tabs = refs[1:1 + nt]
        outs = refs[1 + nt:1 + 2 * nt]
        idx_v = refs[1 + 2 * nt]
        rows = refs[2 + 2 * nt:2 + 3 * nt]
        sems = refs[2 + 3 * nt:]
        wid = lax.axis_index("s") * info.num_cores + lax.axis_index("c")
        base = wid * per_w

        def body(c, _):
            off = base + c * _SC_CHUNK
            pltpu.sync_copy(idx_h.at[pl.ds(off, _SC_CHUNK)], idx_v)
            cps = [pltpu.async_copy(tabs[i].at[idx_v], rows[i], sems[i])
                   for i in range(nt)]
            for cp in cps:
                cp.wait()
            for i in range(nt):
                pltpu.sync_copy(rows[i], outs[i].at[pl.ds(off, _SC_CHUNK)])
            return 0

        lax.fori_loop(0, nch, body, 0)

    return gat(idx_flat, *tables)


def _layer_kernel(first, *refs):
    if first:
        (x_ref, xj_ref, ef_ref, env_ref, u0_ref, u1_ref, u2_ref,
         wq_ref, wk_ref, wv_ref, wo_ref, xo_ref, vo_ref) = refs
        vec_ref = vj_ref = None
    else:
        (x_ref, vec_ref, xj_ref, vj_ref, ef_ref, env_ref, u0_ref, u1_ref,
         u2_ref, wq_ref, wk_ref, wv_ref, wo_ref, xo_ref, vo_ref) = refs

    xb = x_ref[...]
    q = jnp.dot(xb, wq_ref[...], preferred_element_type=F32)
    wk_ = wk_ref[...]
    wv_ = wv_ref[...]
    env = env_ref[...]

    hs = lax.broadcasted_iota(jnp.int32, (HIDDEN, NHEADS), 0) // 16
    hh = lax.broadcasted_iota(jnp.int32, (HIDDEN, NHEADS), 1)
    Hsum = (hs == hh).astype(F32)
    es = lax.broadcasted_iota(jnp.int32, (NHEADS, HIDDEN), 0)
    eh = lax.broadcasted_iota(jnp.int32, (NHEADS, HIDDEN), 1) // 16
    Hexp = (es == eh).astype(F32)

    ds = jnp.zeros((BLK, HIDDEN), F32)
    dv = [jnp.zeros((BLK, HIDDEN), F32) for _ in range(3)]
    urefs = (u0_ref, u1_ref, u2_ref)
    for k in range(KNN):
        xk = xj_ref[k]
        kk = jnp.dot(xk, wk_, preferred_element_type=F32)
        vk = jnp.dot(xk, wv_, preferred_element_type=F32)
        t = q * kk * ef_ref[k]
        logits = jnp.dot(t, Hsum, preferred_element_type=F32,
                         precision=lax.Precision.HIGHEST)
        a = logits * jax.nn.sigmoid(logits) * env[:, k:k + 1]
        val = vk * jnp.dot(a, Hexp, preferred_element_type=F32,
                           precision=lax.Precision.HIGHEST)
        ds = ds + val
        valb = val.astype(jnp.bfloat16).astype(F32)
        if not first:
            vj = vj_ref[k]
        for d in range(3):
            ub = urefs[d][:, k:k + 1].astype(jnp.bfloat16).astype(F32)
            acc = valb * ub
            if not first:
                acc = acc + val * vj[:, d * HIDDEN:(d + 1) * HIDDEN]
            dv[d] = dv[d] + acc

    xo_ref[...] = xb + jnp.dot(ds, wo_ref[...], preferred_element_type=F32)
    for d in range(3):
        tot = dv[d]
        if not first:
            tot = tot + vec_ref[:, d * HIDDEN:(d + 1) * HIDDEN]
        vo_ref[:, d * HIDDEN:(d + 1) * HIDDEN] = tot


def _layer(x, vec, xj, vj, ef3, env, u0, u1, u2, wq, wk, wv, wo, first):
    full = lambda s: pl.BlockSpec(s, lambda i: tuple(0 for _ in s))
    row = lambda w: pl.BlockSpec((BLK, w), lambda i: (i, 0))
    edge = lambda w: pl.BlockSpec((KNN, BLK, w), lambda i: (0, i, 0))
    in_specs = [row(HIDDEN)]
    args = [x]
    if not first:
        in_specs.append(row(3 * HIDDEN)); args.append(vec)
    in_specs.append(edge(HIDDEN)); args.append(xj)
    if not first:
        in_specs.append(edge(3 * HIDDEN)); args.append(vj)
    in_specs += [edge(HIDDEN), row(KNN), row(KNN), row(KNN), row(KNN),
                 full((HIDDEN, HIDDEN)), full((HIDDEN, HIDDEN)),
                 full((HIDDEN, HIDDEN)), full((HIDDEN, HIDDEN))]
    args += [ef3, env, u0, u1, u2, wq, wk, wv, wo]
    return pl.pallas_call(
        functools.partial(_layer_kernel, first),
        grid=(GRID,),
        in_specs=in_specs,
        out_specs=[row(HIDDEN), row(3 * HIDDEN)],
        out_shape=[jax.ShapeDtypeStruct((NP, HIDDEN), F32),
                   jax.ShapeDtypeStruct((NP, 3 * HIDDEN), F32)],
    )(*args)


def _head_kernel(x_ref, w1_ref, b1_ref, w2_ref, b2_ref, o_ref):
    h1 = _silu(jnp.dot(x_ref[...], w1_ref[...], preferred_element_type=F32)
               + b1_ref[...])
    o_ref[...] = jnp.dot(h1, w2_ref[...], preferred_element_type=F32) + b2_ref[...]


def _head(x, w1, b1, w2, b2):
    rows = x.shape[0]
    nf = w2.shape[1]
    grid = rows // 512
    full = lambda s: pl.BlockSpec(s, lambda i: (0, 0))
    return pl.pallas_call(
        _head_kernel,
        grid=(grid,),
        in_specs=[pl.BlockSpec((512, HIDDEN), lambda i: (i, 0)),
                  full((HIDDEN, HIDDEN)), full((1, HIDDEN)),
                  full((HIDDEN, nf)), full((1, nf))],
        out_specs=pl.BlockSpec((512, nf), lambda i: (i, 0)),
        out_shape=jax.ShapeDtypeStruct((rows, nf), F32),
    )(x, w1, b1, w2, b2)


def _velseg_kernel(vec_ref, w8_ref, bc_ref, vel_ref, acc_ref):
    vel = jnp.dot(vec_ref[...], w8_ref[...], preferred_element_type=F32)
    lanes = lax.broadcasted_iota(jnp.int32, (BLK, 8), 1)
    vel = jnp.where(lanes == 3, 1.0, vel)
    vel_ref[...] = vel
    seg = lax.broadcasted_iota(jnp.int32, (NBATCH, BLK), 0)
    oh = (seg == bc_ref[...]).astype(F32)
    psum = jnp.dot(oh, vel, preferred_element_type=F32,
                   precision=lax.Precision.HIGHEST)

    @pl.when(pl.program_id(0) == 0)
    def _():
        acc_ref[...] = jnp.zeros_like(acc_ref)

    acc_ref[...] += psum


def _velseg(vec, w8, batch_col):
    full = lambda s: pl.BlockSpec(s, lambda i: tuple(0 for _ in s))
    return pl.pallas_call(
        _velseg_kernel,
        grid=(GRID,),
        in_specs=[pl.BlockSpec((BLK, 3 * HIDDEN), lambda i: (i, 0)),
                  full((3 * HIDDEN, 8)),
                  pl.BlockSpec((1, BLK), lambda i: (0, i))],
        out_specs=[pl.BlockSpec((BLK, 8), lambda i: (i, 0)),
                   full((NBATCH, 8))],
        out_shape=[jax.ShapeDtypeStruct((NP, 8), F32),
                   jax.ShapeDtypeStruct((NBATCH, 8), F32)],
        compiler_params=pltpu.CompilerParams(
            dimension_semantics=("arbitrary",)),
    )(vec, w8, batch_col)


def _meansub_kernel(vel_ref, acc_ref, br_ref, o_ref):
    acc = acc_ref[...]
    cnt = jnp.maximum(acc[:, 3:4], 1.0)
    mean = acc / cnt
    seg = lax.broadcasted_iota(jnp.int32, (BLK, NBATCH), 1)
    oh = (seg == br_ref[...]).astype(F32)
    o_ref[...] = vel_ref[...] - jnp.dot(oh, mean, preferred_element_type=F32,
                                        precision=lax.Precision.HIGHEST)


def _meansub(vel, acc, batch_row):
    full = lambda s: pl.BlockSpec(s, lambda i: tuple(0 for _ in s))
    return pl.pallas_call(
        _meansub_kernel,
        grid=(GRID,),
        in_specs=[pl.BlockSpec((BLK, 8), lambda i: (i, 0)),
                  full((NBATCH, 8)),
                  pl.BlockSpec((BLK, 1), lambda i: (i, 0))],
        out_specs=pl.BlockSpec((BLK, 8), lambda i: (i, 0)),
        out_shape=jax.ShapeDtypeStruct((NP, 8), F32),
    )(vel, acc, batch_row)


def kernel(xh_atoms, xh_residues, t, mask_atoms, mask_residues, params):
    p = params
    tb = t.reshape(1, 1)
    we = p['emb_w']
    eb = p['emb_b'].reshape(1, HIDDEN)
    h_a = jnp.pad(xh_atoms[:, 3:], ((0, 4096 - N_ATOMS), (0, 0)))
    h_r = jnp.pad(xh_residues[:, 3:], ((0, 2048 - N_RES), (0, 0)))
    x_a = _encode(h_a, p['ae_w1'], p['ae_b1'].reshape(1, -1),
                  p['ae_w2'], p['ae_b2'].reshape(1, -1), we, tb, eb)
    x_r = _encode(h_r, p['re_w1'], p['re_b1'].reshape(1, -1),
                  p['re_w2'], p['re_b2'].reshape(1, -1), we, tb, eb)
    x = jnp.concatenate([x_a[:N_ATOMS], x_r[:N_RES],
                         jnp.zeros((NP - N, HIDDEN), F32)], axis=0)

    pos = jnp.concatenate([xh_atoms[:, :3], xh_residues[:, :3]], axis=0)
    pos_pad = jnp.zeros((NP, 8), F32).at[:N, :3].set(pos)
    posT = pos_pad.T
    batch = jnp.concatenate(
        [mask_atoms.astype(jnp.int32), mask_residues.astype(jnp.int32),
         jnp.full((NP - N,), -1, jnp.int32)], axis=0)

    sq_pad = jnp.pad(jnp.sum(pos * pos, axis=-1), (0, NP - N))
    idx, env, u0, u1, u2, ef3 = _knn(pos_pad, posT, sq_pad, batch, p['edge_w'])
    idx_flat = idx.T.reshape(E)

    vec = None
    for l in range(NLAYERS):
        if l == 0:
            (xj,) = _sc_gather(idx_flat, (x,))
            xj3 = xj.reshape(KNN, NP, HIDDEN)
            x, vec = _layer(x, None, xj3, None, ef3, env, u0, u1, u2,
                            p['wq'][l], p['wk'][l], p['wv'][l], p['wo'][l],
                            True)
        else:
            xj, vj = _sc_gather(idx_flat, (x, vec))
            xj3 = xj.reshape(KNN, NP, HIDDEN)
            vj3 = vj.reshape(KNN, NP, 3 * HIDDEN)
            x, vec = _layer(x, vec, xj3, vj3, ef3, env, u0, u1, u2,
                            p['wq'][l], p['wk'][l], p['wv'][l], p['wo'][l],
                            False)

    w8 = jnp.zeros((3 * HIDDEN, 8), F32)
    for d in range(3):
        w8 = w8.at[d * HIDDEN:(d + 1) * HIDDEN, d].set(p['vel_w'][:, 0])
    vel, acc = _velseg(vec, w8, batch.reshape(1, NP))
    velc = _meansub(vel, acc, batch.reshape(NP, 1))

    hf_a = _head(x[:4096], p['ad_w1'], p['ad_b1'].reshape(1, -1),
                 p['ad_w2'], p['ad_b2'].reshape(1, -1))
    x_res = lax.slice(x, (N_ATOMS, 0), (N_ATOMS + 2048, HIDDEN))
    hf_r = _head(x_res, p['rd_w1'], p['rd_b1'].reshape(1, -1),
                 p['rd_w2'], p['rd_b2'].reshape(1, -1))

    atoms_out = jnp.concatenate([velc[:N_ATOMS, :3], hf_a[:N_ATOMS]], axis=-1)
    res_out = jnp.concatenate([velc[N_ATOMS:N, :3], hf_r[:N_RES]], axis=-1)
    return atoms_out, res_out


# speedup vs baseline: 3.5531x; 1.0216x over previous
import functools

import jax
import jax.numpy as jnp
import numpy as np
from jax import lax
from jax.experimental import pallas as pl
from jax.experimental.pallas import tpu as pltpu
from jax.experimental.pallas import tpu_sc as plsc

N_ATOMS = 4000
N_RES = 2000
ATOM_NF = 16
RES_NF = 21
HIDDEN = 128
NLAYERS = 4
NHEADS = 8
NRBF = 32
KNN = 16
CUTOFF = 5.0
NBATCH = 32

N = N_ATOMS + N_RES
NP = 6144
E = NP * KNN
BLK = 256
GRID = NP // BLK

_START = float(np.exp(-CUTOFF))
_BETA = float(((2.0 / NRBF) * (1.0 - _START)) ** -2)
_MEANS = np.linspace(_START, 1.0, NRBF, dtype=np.float32)

F32 = jnp.float32


def _silu(x):
    return x * jax.nn.sigmoid(x)


def _enc_kernel(h_ref, w1_ref, b1_ref, w2_ref, b2_ref, we_ref, tv_ref,
                eb_ref, o_ref):
    h = h_ref[...]
    h1 = _silu(jnp.dot(h, w1_ref[...], preferred_element_type=F32) + b1_ref[...])
    h2 = jnp.dot(h1, w2_ref[...], preferred_element_type=F32) + b2_ref[...]
    h2t = jnp.concatenate([h2, jnp.full((h2.shape[0], 1), tv_ref[0, 0], F32)],
                          axis=1)
    o_ref[...] = jnp.dot(h2t, we_ref[...], preferred_element_type=F32) + eb_ref[...]


def _encode(h, w1, b1, w2, b2, we, tb, eb):
    rows, nf = h.shape
    grid = rows // 512
    full = lambda s: pl.BlockSpec(s, lambda i: (0, 0))
    return pl.pallas_call(
        _enc_kernel,
        grid=(grid,),
        in_specs=[
            pl.BlockSpec((512, nf), lambda i: (i, 0)),
            full((nf, HIDDEN)), full((1, HIDDEN)),
            full((HIDDEN, HIDDEN)), full((1, HIDDEN)),
            full((HIDDEN + 1, HIDDEN)),
            full((1, 1)), full((1, HIDDEN)),
        ],
        out_specs=pl.BlockSpec((512, HIDDEN), lambda i: (i, 0)),
        out_shape=jax.ShapeDtypeStruct((rows, HIDDEN), F32),
    )(h, w1, b1, w2, b2, we, tb, eb)


def _knn_kernel(pos_ref, posT_ref, sqr_ref, sqc_ref, br_ref, bc_ref, ew_ref,
                mean_ref, idx_ref, env_ref, u0_ref, u1_ref, u2_ref, ef_ref,
                d2_ref):
    pos_b = pos_ref[...]
    posT = posT_ref[...]
    sq_col = sqc_ref[...]
    sq_row = sqr_ref[...]
    d2 = sq_row + sq_col - 2.0 * jnp.dot(pos_b, posT,
                                         preferred_element_type=F32)
    d2 = jnp.maximum(d2, 0.0)
    same = br_ref[...] == bc_ref[...]
    d2_ref[...] = jnp.where(same, d2, 1e12)

    col = lax.broadcasted_iota(jnp.int32, (BLK, NP), 1)
    ew = ew_ref[...]
    means = mean_ref[...]
    for k in range(KNN):
        cur = d2_ref[...]
        m = jnp.min(cur, axis=1, keepdims=True)
        cand = jnp.where(cur == m, col, NP)
        j = jnp.min(cand, axis=1, keepdims=True)
        sel = col == j
        d2_ref[...] = jnp.where(sel, 1e30, cur)
        posj = [jnp.sum(jnp.where(sel, posT_ref[d:d + 1, :], 0.0),
                        axis=1, keepdims=True) for d in range(3)]
        dist = jnp.sqrt(jnp.maximum(m, 1e-12))
        env = jnp.where(dist < CUTOFF,
                        0.5 * (jnp.cos(jnp.pi * dist / CUTOFF) + 1.0), 0.0)
        idx_ref[:, k:k + 1] = j
        env_ref[:, k:k + 1] = env
        inv = 1.0 / (dist + 1e-8)
        u0_ref[:, k:k + 1] = (posj[0] - pos_b[:, 0:1]) * inv
        u1_ref[:, k:k + 1] = (posj[1] - pos_b[:, 1:2]) * inv
        u2_ref[:, k:k + 1] = (posj[2] - pos_b[:, 2:3]) * inv
        rbf = jnp.exp(-_BETA * (jnp.exp(-dist) - means) ** 2) * env
        ef_ref[k] = jnp.dot(rbf, ew, preferred_element_type=F32)


def _knn(pos_pad, posT, sq_pad, batch_col, edge_w):
    batch_row = batch_col.reshape(NP, 1)
    full = lambda s: pl.BlockSpec(s, lambda i: tuple(0 for _ in s))
    return pl.pallas_call(
        _knn_kernel,
        grid=(GRID,),
        in_specs=[
            pl.BlockSpec((BLK, 8), lambda i: (i, 0)),
            full((8, NP)),
            pl.BlockSpec((BLK, 1), lambda i: (i, 0)),
            full((1, NP)),
            pl.BlockSpec((BLK, 1), lambda i: (i, 0)),
            full((1, NP)),
            full((NRBF, HIDDEN)),
            full((1, NRBF)),
        ],
        out_specs=[
            pl.BlockSpec((BLK, KNN), lambda i: (i, 0)),
            pl.BlockSpec((BLK, KNN), lambda i: (i, 0)),
            pl.BlockSpec((BLK, KNN), lambda i: (i, 0)),
            pl.BlockSpec((BLK, KNN), lambda i: (i, 0)),
            pl.BlockSpec((BLK, KNN), lambda i: (i, 0)),
            pl.BlockSpec((KNN, BLK, HIDDEN), lambda i: (0, i, 0)),
        ],
        out_shape=[
            jax.ShapeDtypeStruct((NP, KNN), jnp.int32),
            jax.ShapeDtypeStruct((NP, KNN), F32),
            jax.ShapeDtypeStruct((NP, KNN), F32),
            jax.ShapeDtypeStruct((NP, KNN), F32),
            jax.ShapeDtypeStruct((NP, KNN), F32),
            jax.ShapeDtypeStruct((KNN, NP, HIDDEN), F32),
        ],
        scratch_shapes=[pltpu.VMEM((BLK, NP), F32)],
    )(pos_pad, posT, sq_pad.reshape(NP, 1), sq_pad.reshape(1, NP),
      batch_row, batch_col.reshape(1, NP), edge_w,
      jnp.linspace(_START, 1.0, NRBF).astype(F32).reshape(1, NRBF))


_SC_CHUNK = 96


def _sc_gather(idx_flat, tables):
    info = plsc.get_sparse_core_info()
    nw = info.num_cores * info.num_subcores
    per_w = E // nw
    nch = per_w // _SC_CHUNK
    mesh = plsc.VectorSubcoreMesh(core_axis_name="c", subcore_axis_name="s")
    dims = [t.shape[1] for t in tables]

    out_type = tuple(jax.ShapeDtypeStruct((E, d), F32) for d in dims)
    scratch = [pltpu.VMEM((2, _SC_CHUNK), jnp.int32)]
    scratch += [pltpu.VMEM((2, _SC_CHUNK, d), F32) for d in dims]
    scratch += [pltpu.SemaphoreType.DMA for _ in dims for _ in range(2)]

    @functools.partial(pl.kernel, mesh=mesh, out_type=out_type,
                       scratch_types=scratch)
    def gat(*refs):
        nt = len(dims)
        idx_h = refs[0]
        tabs = refs[1:1 + nt]
        outs = refs[1 + nt:1 + 2 * nt]
        idx_v = refs[1 + 2 * nt]
        rows = refs[2 + 2 * nt:2 + 3 * nt]
        sems = refs[2 + 3 * nt:]
        wid = lax.axis_index("s") * info.num_cores + lax.axis_index("c")
        base = wid * per_w

        def start(c, b):
            pltpu.sync_copy(idx_h.at[pl.ds(base + c * _SC_CHUNK, _SC_CHUNK)],
                            idx_v.at[b])
            for i in range(nt):
                pltpu.async_copy(tabs[i].at[idx_v.at[b]], rows[i].at[b],
                                 sems[2 * i + b])

        def drain(c, b):
            for i in range(nt):
                pltpu.make_async_copy(tabs[i].at[idx_v.at[b]], rows[i].at[b],
                                      sems[2 * i + b]).wait()
                pltpu.sync_copy(rows[i].at[b],
                                outs[i].at[pl.ds(base + c * _SC_CHUNK,
                                                 _SC_CHUNK)])

        start(0, 0)

        def body(g, _):
            start(g + 1, 1)
            drain(g, 0)

            @pl.when(g + 2 < nch)
            def _():
                start(g + 2, 0)

            drain(g + 1, 1)
            return 0

        lax.fori_loop(0, nch // 2, lambda p, _: body(2 * p, _), 0)

    return gat(idx_flat, *tables)


def _layer_kernel(first, *refs):
    if first:
        (x_ref, xj_ref, ef_ref, env_ref, u0_ref, u1_ref, u2_ref,
         wq_ref, wk_ref, wv_ref, wo_ref, xo_ref, vo_ref) = refs
        vec_ref = vj_ref = None
    else:
        (x_ref, vec_ref, xj_ref, vj_ref, ef_ref, env_ref, u0_ref, u1_ref,
         u2_ref, wq_ref, wk_ref, wv_ref, wo_ref, xo_ref, vo_ref) = refs

    xb = x_ref[...]
    q = jnp.dot(xb, wq_ref[...], preferred_element_type=F32)
    wk_ = wk_ref[...]
    wv_ = wv_ref[...]
    env = env_ref[...]

    hs = lax.broadcasted_iota(jnp.int32, (HIDDEN, NHEADS), 0) // 16
    hh = lax.broadcasted_iota(jnp.int32, (HIDDEN, NHEADS), 1)
    Hsum = (hs == hh).astype(F32)
    es = lax.broadcasted_iota(jnp.int32, (NHEADS, HIDDEN), 0)
    eh = lax.broadcasted_iota(jnp.int32, (NHEADS, HIDDEN), 1) // 16
    Hexp = (es == eh).astype(F32)

    ds = jnp.zeros((BLK, HIDDEN), F32)
    dv = [jnp.zeros((BLK, HIDDEN), F32) for _ in range(3)]
    urefs = (u0_ref, u1_ref, u2_ref)
    for k in range(KNN):
        xk = xj_ref[k]
        kk = jnp.dot(xk, wk_, preferred_element_type=F32)
        vk = jnp.dot(xk, wv_, preferred_element_type=F32)
        t = q * kk * ef_ref[k]
        logits = jnp.dot(t, Hsum, preferred_element_type=F32,
                         precision=lax.Precision.HIGHEST)
        a = logits * jax.nn.sigmoid(logits) * env[:, k:k + 1]
        val = vk * jnp.dot(a, Hexp, preferred_element_type=F32,
                           precision=lax.Precision.HIGHEST)
        ds = ds + val
        valb = val.astype(jnp.bfloat16).astype(F32)
        if not first:
            vj = vj_ref[k]
        for d in range(3):
            ub = urefs[d][:, k:k + 1].astype(jnp.bfloat16).astype(F32)
            acc = valb * ub
            if not first:
                acc = acc + val * vj[:, d * HIDDEN:(d + 1) * HIDDEN]
            dv[d] = dv[d] + acc

    xo_ref[...] = xb + jnp.dot(ds, wo_ref[...], preferred_element_type=F32)
    for d in range(3):
        tot = dv[d]
        if not first:
            tot = tot + vec_ref[:, d * HIDDEN:(d + 1) * HIDDEN]
        vo_ref[:, d * HIDDEN:(d + 1) * HIDDEN] = tot


def _layer(x, vec, xj, vj, ef3, env, u0, u1, u2, wq, wk, wv, wo, first):
    full = lambda s: pl.BlockSpec(s, lambda i: tuple(0 for _ in s))
    row = lambda w: pl.BlockSpec((BLK, w), lambda i: (i, 0))
    edge = lambda w: pl.BlockSpec((KNN, BLK, w), lambda i: (0, i, 0))
    in_specs = [row(HIDDEN)]
    args = [x]
    if not first:
        in_specs.append(row(3 * HIDDEN)); args.append(vec)
    in_specs.append(edge(HIDDEN)); args.append(xj)
    if not first:
        in_specs.append(edge(3 * HIDDEN)); args.append(vj)
    in_specs += [edge(HIDDEN), row(KNN), row(KNN), row(KNN), row(KNN),
                 full((HIDDEN, HIDDEN)), full((HIDDEN, HIDDEN)),
                 full((HIDDEN, HIDDEN)), full((HIDDEN, HIDDEN))]
    args += [ef3, env, u0, u1, u2, wq, wk, wv, wo]
    return pl.pallas_call(
        functools.partial(_layer_kernel, first),
        grid=(GRID,),
        in_specs=in_specs,
        out_specs=[row(HIDDEN), row(3 * HIDDEN)],
        out_shape=[jax.ShapeDtypeStruct((NP, HIDDEN), F32),
                   jax.ShapeDtypeStruct((NP, 3 * HIDDEN), F32)],
    )(*args)


def _head_kernel(x_ref, w1_ref, b1_ref, w2_ref, b2_ref, o_ref):
    h1 = _silu(jnp.dot(x_ref[...], w1_ref[...], preferred_element_type=F32)
               + b1_ref[...])
    o_ref[...] = jnp.dot(h1, w2_ref[...], preferred_element_type=F32) + b2_ref[...]


def _head(x, w1, b1, w2, b2):
    rows = x.shape[0]
    nf = w2.shape[1]
    grid = rows // 512
    full = lambda s: pl.BlockSpec(s, lambda i: (0, 0))
    return pl.pallas_call(
        _head_kernel,
        grid=(grid,),
        in_specs=[pl.BlockSpec((512, HIDDEN), lambda i: (i, 0)),
                  full((HIDDEN, HIDDEN)), full((1, HIDDEN)),
                  full((HIDDEN, nf)), full((1, nf))],
        out_specs=pl.BlockSpec((512, nf), lambda i: (i, 0)),
        out_shape=jax.ShapeDtypeStruct((rows, nf), F32),
    )(x, w1, b1, w2, b2)


def _velseg_kernel(vec_ref, w8_ref, bc_ref, vel_ref, acc_ref):
    vel = jnp.dot(vec_ref[...], w8_ref[...], preferred_element_type=F32)
    lanes = lax.broadcasted_iota(jnp.int32, (BLK, 8), 1)
    vel = jnp.where(lanes == 3, 1.0, vel)
    vel_ref[...] = vel
    seg = lax.broadcasted_iota(jnp.int32, (NBATCH, BLK), 0)
    oh = (seg == bc_ref[...]).astype(F32)
    psum = jnp.dot(oh, vel, preferred_element_type=F32,
                   precision=lax.Precision.HIGHEST)

    @pl.when(pl.program_id(0) == 0)
    def _():
        acc_ref[...] = jnp.zeros_like(acc_ref)

    acc_ref[...] += psum


def _velseg(vec, w8, batch_col):
    full = lambda s: pl.BlockSpec(s, lambda i: tuple(0 for _ in s))
    return pl.pallas_call(
        _velseg_kernel,
        grid=(GRID,),
        in_specs=[pl.BlockSpec((BLK, 3 * HIDDEN), lambda i: (i, 0)),
                  full((3 * HIDDEN, 8)),
                  pl.BlockSpec((1, BLK), lambda i: (0, i))],
        out_specs=[pl.BlockSpec((BLK, 8), lambda i: (i, 0)),
                   full((NBATCH, 8))],
        out_shape=[jax.ShapeDtypeStruct((NP, 8), F32),
                   jax.ShapeDtypeStruct((NBATCH, 8), F32)],
        compiler_params=pltpu.CompilerParams(
            dimension_semantics=("arbitrary",)),
    )(vec, w8, batch_col)


def _meansub_kernel(vel_ref, acc_ref, br_ref, o_ref):
    acc = acc_ref[...]
    cnt = jnp.maximum(acc[:, 3:4], 1.0)
    mean = acc / cnt
    seg = lax.broadcasted_iota(jnp.int32, (BLK, NBATCH), 1)
    oh = (seg == br_ref[...]).astype(F32)
    o_ref[...] = vel_ref[...] - jnp.dot(oh, mean, preferred_element_type=F32,
                                        precision=lax.Precision.HIGHEST)


def _meansub(vel, acc, batch_row):
    full = lambda s: pl.BlockSpec(s, lambda i: tuple(0 for _ in s))
    return pl.pallas_call(
        _meansub_kernel,
        grid=(GRID,),
        in_specs=[pl.BlockSpec((BLK, 8), lambda i: (i, 0)),
                  full((NBATCH, 8)),
                  pl.BlockSpec((BLK, 1), lambda i: (i, 0))],
        out_specs=pl.BlockSpec((BLK, 8), lambda i: (i, 0)),
        out_shape=jax.ShapeDtypeStruct((NP, 8), F32),
    )(vel, acc, batch_row)


def kernel(xh_atoms, xh_residues, t, mask_atoms, mask_residues, params):
    p = params
    tb = t.reshape(1, 1)
    we = p['emb_w']
    eb = p['emb_b'].reshape(1, HIDDEN)
    h_a = jnp.pad(xh_atoms[:, 3:], ((0, 4096 - N_ATOMS), (0, 0)))
    h_r = jnp.pad(xh_residues[:, 3:], ((0, 2048 - N_RES), (0, 0)))
    x_a = _encode(h_a, p['ae_w1'], p['ae_b1'].reshape(1, -1),
                  p['ae_w2'], p['ae_b2'].reshape(1, -1), we, tb, eb)
    x_r = _encode(h_r, p['re_w1'], p['re_b1'].reshape(1, -1),
                  p['re_w2'], p['re_b2'].reshape(1, -1), we, tb, eb)
    x = jnp.concatenate([x_a[:N_ATOMS], x_r[:N_RES],
                         jnp.zeros((NP - N, HIDDEN), F32)], axis=0)

    pos = jnp.concatenate([xh_atoms[:, :3], xh_residues[:, :3]], axis=0)
    pos_pad = jnp.zeros((NP, 8), F32).at[:N, :3].set(pos)
    posT = pos_pad.T
    batch = jnp.concatenate(
        [mask_atoms.astype(jnp.int32), mask_residues.astype(jnp.int32),
         jnp.full((NP - N,), -1, jnp.int32)], axis=0)

    sq_pad = jnp.pad(jnp.sum(pos * pos, axis=-1), (0, NP - N))
    idx, env, u0, u1, u2, ef3 = _knn(pos_pad, posT, sq_pad, batch, p['edge_w'])
    idx_flat = idx.T.reshape(E)

    vec = None
    for l in range(NLAYERS):
        if l == 0:
            (xj,) = _sc_gather(idx_flat, (x,))
            xj3 = xj.reshape(KNN, NP, HIDDEN)
            x, vec = _layer(x, None, xj3, None, ef3, env, u0, u1, u2,
                            p['wq'][l], p['wk'][l], p['wv'][l], p['wo'][l],
                            True)
        else:
            xj, vj = _sc_gather(idx_flat, (x, vec))
            xj3 = xj.reshape(KNN, NP, HIDDEN)
            vj3 = vj.reshape(KNN, NP, 3 * HIDDEN)
            x, vec = _layer(x, vec, xj3, vj3, ef3, env, u0, u1, u2,
                            p['wq'][l], p['wk'][l], p['wv'][l], p['wo'][l],
                            False)

    w8 = jnp.zeros((3 * HIDDEN, 8), F32)
    for d in range(3):
        w8 = w8.at[d * HIDDEN:(d + 1) * HIDDEN, d].set(p['vel_w'][:, 0])
    vel, acc = _velseg(vec, w8, batch.reshape(1, NP))
    velc = _meansub(vel, acc, batch.reshape(NP, 1))

    hf_a = _head(x[:4096], p['ad_w1'], p['ad_b1'].reshape(1, -1),
                 p['ad_w2'], p['ad_b2'].reshape(1, -1))
    x_res = lax.slice(x, (N_ATOMS, 0), (N_ATOMS + 2048, HIDDEN))
    hf_r = _head(x_res, p['rd_w1'], p['rd_b1'].reshape(1, -1),
                 p['rd_w2'], p['rd_b2'].reshape(1, -1))

    atoms_out = jnp.concatenate([velc[:N_ATOMS, :3], hf_a[:N_ATOMS]], axis=-1)
    res_out = jnp.concatenate([velc[N_ATOMS:N, :3], hf_r[:N_RES]], axis=-1)
    return atoms_out, res_out
